# baseline (speedup 1.0000x reference)
import functools

import jax
import jax.numpy as jnp
from jax import lax
from jax.experimental import pallas as pl
from jax.experimental.pallas import tpu as pltpu
from jax.experimental.pallas import tpu_sc as plsc

B = 16
P = 65536
NB = 8
C = NB ** 3
F = 10
TBL = C * F
NW = 32
HALF = P // 2
CHUNK = 4096
NCH = HALF // CHUNK


def _bbox_body(x_ref, o_ref):
    x = x_ref[...]
    mx = jnp.max(jnp.abs(x), axis=(0, 2))
    o_ref[...] = jnp.broadcast_to(mx[:, None], (B, 128))


def _bbox_max(xp):
    return pl.pallas_call(
        _bbox_body,
        grid=(1,),
        in_specs=[pl.BlockSpec((3, B, P), lambda i: (0, 0, 0))],
        out_specs=pl.BlockSpec((B, 128), lambda i: (0, 0)),
        out_shape=jax.ShapeDtypeStruct((B, 128), jnp.float32),
    )(xp)


_mesh = plsc.VectorSubcoreMesh(core_axis_name="c", subcore_axis_name="s")


@functools.partial(
    pl.kernel,
    mesh=_mesh,
    out_type=jax.ShapeDtypeStruct((NW, TBL), jnp.float32),
    scratch_types=[
        pltpu.VMEM((2, 6, CHUNK), jnp.float32),
        pltpu.VMEM((TBL,), jnp.float32),
        pltpu.VMEM((16,), jnp.float32),
        pltpu.SemaphoreType.DMA,
        pltpu.SemaphoreType.DMA,
    ],
    compiler_params=pltpu.CompilerParams(needs_layout_passes=False),
)
def _sc_hist(xp_hbm, maxes_hbm, out_hbm, bufs, table, mvec, sem0, sem1):
    c = lax.axis_index("c")
    s = lax.axis_index("s")
    wid = s * 2 + c
    base = c * HALF

    pltpu.sync_copy(maxes_hbm, mvec)
    m = plsc.load_gather(mvec, [jnp.full((16,), s, jnp.int32)])
    t = jnp.maximum(2.0 * m, 1e-5)
    rt8 = 8.0 / t

    def zbody(i, carry):
        table[pl.ds(i * 16, 16)] = jnp.zeros((16,), jnp.float32)
        return carry

    lax.fori_loop(0, TBL // 16, zbody, 0)

    sems = (sem0, sem1)

    def start(g, slot):
        for ch in range(6):
            pltpu.async_copy(
                xp_hbm.at[ch, s, pl.ds(base + g * CHUNK, CHUNK)],
                bufs.at[slot, ch], sems[slot])

    def wait(g, slot):
        for ch in range(6):
            pltpu.make_async_copy(
                xp_hbm.at[ch, s, pl.ds(base + g * CHUNK, CHUNK)],
                bufs.at[slot, ch], sems[slot]).wait()

    ones = jnp.ones((16,), jnp.float32)

    UNROLL = 8

    def process(slot):
        def one(o):
            x = bufs[slot, 0, o]
            y = bufs[slot, 1, o]
            z = bufs[slot, 2, o]
            ox = bufs[slot, 3, o]
            oy = bufs[slot, 4, o]
            oz = bufs[slot, 5, o]
            px8 = (x + m) * rt8
            py8 = (y + m) * rt8
            pz8 = (z + m) * rt8
            cx = jnp.minimum(px8.astype(jnp.int32), 7)
            cy = jnp.minimum(py8.astype(jnp.int32), 7)
            cz = jnp.minimum(pz8.astype(jnp.int32), 7)
            cell = (cx * 8 + cy) * 8 + cz
            dx = (px8 - cx.astype(jnp.float32) - 0.5) * 0.125
            dy = (py8 - cy.astype(jnp.float32) - 0.5) * 0.125
            dz = (pz8 - cz.astype(jnp.float32) - 0.5) * 0.125
            plsc.addupdate_scatter(table, [cell], ones)
            plsc.addupdate_scatter(table, [cell + 512], dx)
            plsc.addupdate_scatter(table, [cell + 1024], dy)
            plsc.addupdate_scatter(table, [cell + 1536], dz)
            plsc.addupdate_scatter(table, [cell + 2048], ox * ox)
            plsc.addupdate_scatter(table, [cell + 2560], ox * oy)
            plsc.addupdate_scatter(table, [cell + 3072], ox * oz)
            plsc.addupdate_scatter(table, [cell + 3584], oy * oy)
            plsc.addupdate_scatter(table, [cell + 4096], oy * oz)
            plsc.addupdate_scatter(table, [cell + 4608], oz * oz)

        @plsc.parallel_loop(0, CHUNK // 16, unroll=UNROLL)
        def gbody(i):
            one(pl.ds(i * 16, 16))

    start(0, 0)
    for g in range(NCH):
        if g + 1 < NCH:
            start(g + 1, (g + 1) % 2)
        wait(g, g % 2)
        process(g % 2)

    pltpu.sync_copy(table, out_hbm.at[wid])


def _post_body(p_ref, o_ref):
    t = p_ref[0, 0] + p_ref[0, 1]
    freq = t[0:1]
    fmax = jnp.maximum(freq, 1.0)
    up = 1.0 / jnp.sqrt(fmax)
    ffreq = 0.001 * freq * up
    fmean = t[1:4] * up
    fcov = t[4:10] / fmax
    s = jnp.concatenate([ffreq, fmean, fcov], axis=0)
    norm = jnp.sqrt(jnp.sum(s * s))
    o_ref[0] = s / jnp.maximum(norm, 1e-12)


def _post(partials):
    p = partials.reshape(B, 2, F, C)
    return pl.pallas_call(
        _post_body,
        grid=(B,),
        in_specs=[pl.BlockSpec((1, 2, F, C), lambda b: (b, 0, 0, 0))],
        out_specs=pl.BlockSpec((1, F, C), lambda b: (b, 0, 0)),
        out_shape=jax.ShapeDtypeStruct((B, F, C), jnp.float32),
    )(p)


def kernel(input):
    xp = jnp.transpose(input, (2, 0, 1))
    maxes = _bbox_max(xp)[:, 0]
    partials = _sc_hist(xp, maxes)
    out = _post(partials)
    return out.transpose(0, 2, 1).reshape(B, C * F)


# speedup vs baseline: 1.0179x; 1.0179x over previous
import functools

import jax
import jax.numpy as jnp
from jax import lax
from jax.experimental import pallas as pl
from jax.experimental.pallas import tpu as pltpu
from jax.experimental.pallas import tpu_sc as plsc

B = 16
P = 65536
NB = 8
C = NB ** 3
F = 10
TBL = C * F
NW = 32
HALF = P // 2
CHUNK = 4096
NCH = HALF // CHUNK


def _bbox_body(x_ref, o_ref):
    x = x_ref[...]
    mx = jnp.max(jnp.abs(x), axis=(0, 2))
    o_ref[...] = jnp.broadcast_to(mx[:, None], (B, 128))


def _bbox_max(xp):
    return pl.pallas_call(
        _bbox_body,
        grid=(1,),
        in_specs=[pl.BlockSpec((3, B, P), lambda i: (0, 0, 0))],
        out_specs=pl.BlockSpec((B, 128), lambda i: (0, 0)),
        out_shape=jax.ShapeDtypeStruct((B, 128), jnp.float32),
    )(xp)


_mesh = plsc.VectorSubcoreMesh(core_axis_name="c", subcore_axis_name="s")


@functools.partial(
    pl.kernel,
    mesh=_mesh,
    out_type=jax.ShapeDtypeStruct((NW, TBL), jnp.float32),
    scratch_types=[
        pltpu.VMEM((2, 6, CHUNK), jnp.float32),
        pltpu.VMEM((TBL,), jnp.float32),
        pltpu.VMEM((16,), jnp.float32),
        pltpu.SemaphoreType.DMA,
        pltpu.SemaphoreType.DMA,
    ],
    compiler_params=pltpu.CompilerParams(needs_layout_passes=False),
)
def _sc_hist(xp_hbm, maxes_hbm, out_hbm, bufs, table, mvec, sem0, sem1):
    c = lax.axis_index("c")
    s = lax.axis_index("s")
    wid = s * 2 + c
    base = c * HALF

    pltpu.sync_copy(maxes_hbm, mvec)
    m = plsc.load_gather(mvec, [jnp.full((16,), s, jnp.int32)])
    t = jnp.maximum(2.0 * m, 1e-5)
    rt8 = 8.0 / t

    def zbody(i, carry):
        table[pl.ds(i * 16, 16)] = jnp.zeros((16,), jnp.float32)
        return carry

    lax.fori_loop(0, TBL // 16, zbody, 0)

    sems = (sem0, sem1)

    def start(g, slot):
        for ch in range(6):
            pltpu.async_copy(
                xp_hbm.at[ch, s, pl.ds(base + g * CHUNK, CHUNK)],
                bufs.at[slot, ch], sems[slot])

    def wait(g, slot):
        for ch in range(6):
            pltpu.make_async_copy(
                xp_hbm.at[ch, s, pl.ds(base + g * CHUNK, CHUNK)],
                bufs.at[slot, ch], sems[slot]).wait()

    ones = jnp.ones((16,), jnp.float32)

    UNROLL = 4

    def process(slot):
        def one(o):
            x = bufs[slot, 0, o]
            y = bufs[slot, 1, o]
            z = bufs[slot, 2, o]
            ox = bufs[slot, 3, o]
            oy = bufs[slot, 4, o]
            oz = bufs[slot, 5, o]
            px8 = (x + m) * rt8
            py8 = (y + m) * rt8
            pz8 = (z + m) * rt8
            cx = jnp.minimum(px8.astype(jnp.int32), 7)
            cy = jnp.minimum(py8.astype(jnp.int32), 7)
            cz = jnp.minimum(pz8.astype(jnp.int32), 7)
            cell = (cx * 8 + cy) * 8 + cz
            dx = (px8 - cx.astype(jnp.float32) - 0.5) * 0.125
            dy = (py8 - cy.astype(jnp.float32) - 0.5) * 0.125
            dz = (pz8 - cz.astype(jnp.float32) - 0.5) * 0.125
            plsc.addupdate_scatter(table, [cell], ones)
            plsc.addupdate_scatter(table, [cell + 512], dx)
            plsc.addupdate_scatter(table, [cell + 1024], dy)
            plsc.addupdate_scatter(table, [cell + 1536], dz)
            plsc.addupdate_scatter(table, [cell + 2048], ox * ox)
            plsc.addupdate_scatter(table, [cell + 2560], ox * oy)
            plsc.addupdate_scatter(table, [cell + 3072], ox * oz)
            plsc.addupdate_scatter(table, [cell + 3584], oy * oy)
            plsc.addupdate_scatter(table, [cell + 4096], oy * oz)
            plsc.addupdate_scatter(table, [cell + 4608], oz * oz)

        @plsc.parallel_loop(0, CHUNK // 16, unroll=UNROLL)
        def gbody(i):
            one(pl.ds(i * 16, 16))

    start(0, 0)
    for g in range(NCH):
        if g + 1 < NCH:
            start(g + 1, (g + 1) % 2)
        wait(g, g % 2)
        process(g % 2)

    pltpu.sync_copy(table, out_hbm.at[wid])


def _post_body(p_ref, o_ref):
    t = p_ref[0, 0] + p_ref[0, 1]
    freq = t[0:1]
    fmax = jnp.maximum(freq, 1.0)
    up = 1.0 / jnp.sqrt(fmax)
    ffreq = 0.001 * freq * up
    fmean = t[1:4] * up
    fcov = t[4:10] / fmax
    s = jnp.concatenate([ffreq, fmean, fcov], axis=0)
    norm = jnp.sqrt(jnp.sum(s * s))
    o_ref[0] = s / jnp.maximum(norm, 1e-12)


def _post(partials):
    p = partials.reshape(B, 2, F, C)
    return pl.pallas_call(
        _post_body,
        grid=(B,),
        in_specs=[pl.BlockSpec((1, 2, F, C), lambda b: (b, 0, 0, 0))],
        out_specs=pl.BlockSpec((1, F, C), lambda b: (b, 0, 0)),
        out_shape=jax.ShapeDtypeStruct((B, F, C), jnp.float32),
    )(p)


def kernel(input):
    xp = jnp.transpose(input, (2, 0, 1))
    maxes = _bbox_max(xp)[:, 0]
    partials = _sc_hist(xp, maxes)
    out = _post(partials)
    return out.transpose(0, 2, 1).reshape(B, C * F)


# speedup vs baseline: 1.0395x; 1.0212x over previous
import functools

import jax
import jax.numpy as jnp
from jax import lax
from jax.experimental import pallas as pl
from jax.experimental.pallas import tpu as pltpu
from jax.experimental.pallas import tpu_sc as plsc

B = 16
P = 65536
NB = 8
C = NB ** 3
F = 10
TBL = C * F
HREG = TBL // 2
NW = 32
HALF = P // 2
CHUNK = 4096
NCH = HALF // CHUNK


def _bbox_body(x_ref, o_ref):
    x = x_ref[...]
    mx = jnp.max(jnp.abs(x), axis=(0, 2))
    o_ref[...] = jnp.broadcast_to(mx[:, None], (B, 128))


def _bbox_max(xp):
    return pl.pallas_call(
        _bbox_body,
        grid=(1,),
        in_specs=[pl.BlockSpec((3, B, P), lambda i: (0, 0, 0))],
        out_specs=pl.BlockSpec((B, 128), lambda i: (0, 0)),
        out_shape=jax.ShapeDtypeStruct((B, 128), jnp.float32),
    )(xp)


_mesh = plsc.VectorSubcoreMesh(core_axis_name="c", subcore_axis_name="s")


def _rsqrt_newton(x):
    i = plsc.bitcast(x, jnp.int32)
    y = plsc.bitcast(0x5F3759DF - (i >> 1), jnp.float32)
    for _ in range(3):
        y = y * (1.5 - 0.5 * x * y * y)
    return y


_SC_KERNEL_KW = dict(
    mesh=_mesh,
    out_type=jax.ShapeDtypeStruct((B, TBL), jnp.float32),
    scratch_types=[
        pltpu.VMEM((2, 6, CHUNK), jnp.float32),
        pltpu.VMEM((TBL,), jnp.float32),
        pltpu.VMEM((TBL,), jnp.float32),
        pltpu.VMEM((HREG,), jnp.float32),
        pltpu.VMEM((16,), jnp.float32),
        pltpu.VMEM_SHARED((16, TBL), jnp.float32),
        pltpu.SemaphoreType.DMA,
        pltpu.SemaphoreType.DMA,
        pltpu.SemaphoreType.DMA,
    ],
    compiler_params=pltpu.CompilerParams(needs_layout_passes=False),
)


def _sc_body(xp_hbm, maxes_hbm, out_hbm, bufs, table, ptab, aos, mvec,
             shreg, sem0, sem1, sem2):
    c = lax.axis_index("c")
    s = lax.axis_index("s")
    b = c * 8 + s // 2
    h = s % 2
    partner = s + 1 - 2 * h

    pltpu.sync_copy(maxes_hbm, mvec)
    m = plsc.load_gather(mvec, [jnp.full((16,), b, jnp.int32)])
    t = jnp.maximum(2.0 * m, 1e-5)
    rt8 = 8.0 / t

    @plsc.parallel_loop(0, TBL // 16)
    def _zero(i):
        table[pl.ds(i * 16, 16)] = jnp.zeros((16,), jnp.float32)

    sems = (sem0, sem1)
    pbase = h * HALF

    def start(g, slot):
        for ch in range(6):
            pltpu.async_copy(
                xp_hbm.at[ch, b, pl.ds(pbase + g * CHUNK, CHUNK)],
                bufs.at[slot, ch], sems[slot])

    def wait(g, slot):
        for ch in range(6):
            pltpu.make_async_copy(
                xp_hbm.at[ch, b, pl.ds(pbase + g * CHUNK, CHUNK)],
                bufs.at[slot, ch], sems[slot]).wait()

    ones = jnp.ones((16,), jnp.float32)
    UNROLL = 4

    def process(slot):
        def one(o):
            x = bufs[slot, 0, o]
            y = bufs[slot, 1, o]
            z = bufs[slot, 2, o]
            ox = bufs[slot, 3, o]
            oy = bufs[slot, 4, o]
            oz = bufs[slot, 5, o]
            px8 = (x + m) * rt8
            py8 = (y + m) * rt8
            pz8 = (z + m) * rt8
            cx = jnp.minimum(px8.astype(jnp.int32), 7)
            cy = jnp.minimum(py8.astype(jnp.int32), 7)
            cz = jnp.minimum(pz8.astype(jnp.int32), 7)
            cell = (cx * 8 + cy) * 8 + cz
            idx = (cell & 256) * 10 + (cell & 255)
            dx = (px8 - cx.astype(jnp.float32) - 0.5) * 0.125
            dy = (py8 - cy.astype(jnp.float32) - 0.5) * 0.125
            dz = (pz8 - cz.astype(jnp.float32) - 0.5) * 0.125
            plsc.addupdate_scatter(table, [idx], ones)
            plsc.addupdate_scatter(table, [idx + 256], dx)
            plsc.addupdate_scatter(table, [idx + 512], dy)
            plsc.addupdate_scatter(table, [idx + 768], dz)
            plsc.addupdate_scatter(table, [idx + 1024], ox * ox)
            plsc.addupdate_scatter(table, [idx + 1280], ox * oy)
            plsc.addupdate_scatter(table, [idx + 1536], ox * oz)
            plsc.addupdate_scatter(table, [idx + 1792], oy * oy)
            plsc.addupdate_scatter(table, [idx + 2048], oy * oz)
            plsc.addupdate_scatter(table, [idx + 2304], oz * oz)

        @plsc.parallel_loop(0, CHUNK // 16, unroll=UNROLL)
        def gbody(i):
            one(pl.ds(i * 16, 16))

    start(0, 0)
    for g in range(NCH):
        if g + 1 < NCH:
            start(g + 1, (g + 1) % 2)
        wait(g, g % 2)
        process(g % 2)

    pltpu.sync_copy(table, shreg.at[partner])
    plsc.subcore_barrier()
    pltpu.sync_copy(shreg.at[s], ptab)

    @plsc.parallel_loop(0, TBL // 16, unroll=2)
    def _merge(i):
        o = pl.ds(i * 16, 16)
        ptab[o] = ptab[o] + table[o]

    def sbody(i, acc):
        rb = (i // 16) * HREG + (i % 16) * 16
        o = pl.ds(rb, 16)
        freq = ptab[o]
        fmax = jnp.maximum(freq, 1.0)
        up = _rsqrt_newton(fmax)
        rf = up * up
        v0 = 0.001 * freq * up
        ptab[o] = v0
        acc = acc + v0 * v0
        for k in range(1, 4):
            ok = pl.ds(rb + k * 256, 16)
            vk = ptab[ok] * up
            ptab[ok] = vk
            acc = acc + vk * vk
        for k in range(4, 10):
            ok = pl.ds(rb + k * 256, 16)
            vk = ptab[ok] * rf
            ptab[ok] = vk
            acc = acc + vk * vk
        return acc

    acc = lax.fori_loop(0, 32, sbody, jnp.zeros((16,), jnp.float32))
    n2 = jnp.sum(acc, axis=0)
    nrm = n2 * _rsqrt_newton(jnp.full((16,), n2, jnp.float32))
    inv = 1.0 / jnp.maximum(nrm, 1e-12)

    lane = lax.iota(jnp.int32, 16)

    @plsc.parallel_loop(0, HREG // 16, unroll=2)
    def _permute(i):
        j = lane + i * 16
        q = j // 10
        k = j - q * 10
        src = plsc.load_gather(ptab, [h * HREG + k * 256 + q])
        aos[pl.ds(i * 16, 16)] = src * inv

    pltpu.sync_copy(aos, out_hbm.at[b, pl.ds(h * HREG, HREG)])


_sc_hist = pl.kernel(_sc_body, **_SC_KERNEL_KW)


def kernel(input):
    xp = jnp.transpose(input, (2, 0, 1))
    maxes = _bbox_max(xp)[:, 0]
    return _sc_hist(xp, maxes)
